# Initial kernel scaffold; baseline (speedup 1.0000x reference)
import jax
import jax.numpy as jnp
from jax.experimental import pallas as pl


def kernel(x, adj_t, W1, b1, W2, b2, W3, b3):
    raise NotImplementedError("write your pallas kernel here")


# speedup vs baseline: 5.4702x; 5.4702x over previous
import functools

import jax
import jax.numpy as jnp
from jax import lax
from jax.experimental import pallas as pl
from jax.experimental.pallas import tpu as pltpu
from jax.experimental.pallas import tpu_sc as plsc

N_NODES = 10000
N_EDGES = 320000
D = 128

NC = 2
NS = 16
NW = NC * NS

NPAD = 10240
EPAD = 327680
EPT = EPAD // NW
CH = 128
NCHUNK = EPT // CH
RPT = NPAD // NS
RZ = 128
WDEG = 128

def _sc_degree_body(dst_hbm, ones_hbm, zero_hbm, out_hbm, acc, idx_v, ones_v,
                    zb_v, sem):
  cid = lax.axis_index("c")
  sid = lax.axis_index("s")
  pltpu.sync_copy(zero_hbm, zb_v)
  for r in range(RPT // RZ):
    pltpu.sync_copy(zb_v, acc.at[pl.ds(sid * RPT + r * RZ, RZ)])
  pltpu.sync_copy(ones_hbm, ones_v)
  plsc.subcore_barrier()
  ebase = cid * (EPAD // NC) + sid * EPT

  @pl.loop(0, NCHUNK)
  def _(j):
    pltpu.sync_copy(dst_hbm.at[pl.ds(ebase + j * CH, CH)], idx_v)
    pltpu.sync_copy(ones_v, acc.at[idx_v], add=True)

  plsc.subcore_barrier()
  for r in range(RPT // RZ):
    rows = pl.ds(sid * RPT + r * RZ, RZ)
    pltpu.sync_copy(acc.at[rows], zb_v)
    pltpu.sync_copy(zb_v, out_hbm.at[cid, rows])


def _sc_prop_body(h_hbm, src_hbm, dst_hbm, zero_hbm, out_hbm, acc, sidx_v,
                  didx_v, rows_v, sem):
  cid = lax.axis_index("c")
  sid = lax.axis_index("s")
  pltpu.sync_copy(zero_hbm, rows_v)
  for r in range(RPT // RZ):
    pltpu.sync_copy(rows_v, acc.at[pl.ds(sid * RPT + r * RZ, RZ)])
  plsc.subcore_barrier()
  ebase = cid * (EPAD // NC) + sid * EPT

  @pl.loop(0, NCHUNK)
  def _(j):
    pltpu.sync_copy(src_hbm.at[pl.ds(ebase + j * CH, CH)], sidx_v)
    pltpu.sync_copy(dst_hbm.at[pl.ds(ebase + j * CH, CH)], didx_v)
    pltpu.async_copy(h_hbm.at[sidx_v], rows_v, sem).wait()
    pltpu.sync_copy(rows_v, acc.at[didx_v], add=True)

  plsc.subcore_barrier()
  for r in range(RPT // RZ):
    rows = pl.ds(sid * RPT + r * RZ, RZ)
    pltpu.sync_copy(acc.at[rows], rows_v)
    pltpu.sync_copy(rows_v, out_hbm.at[cid, rows])


@functools.cache
def _sc_kernels():
  mesh = plsc.VectorSubcoreMesh(
      core_axis_name="c", subcore_axis_name="s",
      num_cores=NC, num_subcores=NS)
  sc_degree = pl.kernel(
      _sc_degree_body,
      out_type=jax.ShapeDtypeStruct((NC, NPAD, WDEG), jnp.float32),
      mesh=mesh,
      scratch_types=[
          pltpu.VMEM_SHARED((NPAD, WDEG), jnp.float32),
          pltpu.VMEM((CH,), jnp.int32),
          pltpu.VMEM((CH, WDEG), jnp.float32),
          pltpu.VMEM((RZ, WDEG), jnp.float32),
          pltpu.SemaphoreType.DMA,
      ])
  sc_prop = pl.kernel(
      _sc_prop_body,
      out_type=jax.ShapeDtypeStruct((NC, NPAD, D), jnp.float32),
      mesh=mesh,
      scratch_types=[
          pltpu.VMEM_SHARED((NPAD, D), jnp.float32),
          pltpu.VMEM((CH,), jnp.int32),
          pltpu.VMEM((CH,), jnp.int32),
          pltpu.VMEM((RZ, D), jnp.float32),
          pltpu.SemaphoreType.DMA,
      ])
  return sc_degree, sc_prop


BLK = 1024
_GRID = NPAD // BLK


def _dis_of(deg_blk):
  return jnp.where(deg_blk > 0.0,
                   lax.rsqrt(jnp.maximum(deg_blk, 1e-12)), 0.0)


def _tc_first_body(x_ref, w_ref, degp_ref, h_ref, dis_ref):
  deg = degp_ref[0] + degp_ref[1]
  dis = _dis_of(deg)
  h = jnp.dot(x_ref[...], w_ref[...], preferred_element_type=jnp.float32)
  h_ref[...] = dis[:, 0:1] * h
  dis_ref[...] = dis


def _tc_first(x, w, deg_parts):
  return pl.pallas_call(
      _tc_first_body,
      grid=(_GRID,),
      in_specs=[
          pl.BlockSpec((BLK, D), lambda i: (i, 0)),
          pl.BlockSpec((D, D), lambda i: (0, 0)),
          pl.BlockSpec((NC, BLK, WDEG), lambda i: (0, i, 0)),
      ],
      out_specs=[
          pl.BlockSpec((BLK, D), lambda i: (i, 0)),
          pl.BlockSpec((BLK, WDEG), lambda i: (i, 0)),
      ],
      out_shape=[
          jax.ShapeDtypeStruct((NPAD, D), jnp.float32),
          jax.ShapeDtypeStruct((NPAD, WDEG), jnp.float32),
      ],
  )(x, w, deg_parts)


def _tc_mid_body(p_ref, dis_ref, b_ref, w_ref, h_ref):
  d = dis_ref[:, 0:1]
  y = jnp.maximum(d * (p_ref[0] + p_ref[1]) + b_ref[...], 0.0)
  h_ref[...] = d * jnp.dot(y, w_ref[...], preferred_element_type=jnp.float32)


def _tc_mid(parts, dis, b, w):
  return pl.pallas_call(
      _tc_mid_body,
      grid=(_GRID,),
      in_specs=[
          pl.BlockSpec((NC, BLK, D), lambda i: (0, i, 0)),
          pl.BlockSpec((BLK, WDEG), lambda i: (i, 0)),
          pl.BlockSpec((1, D), lambda i: (0, 0)),
          pl.BlockSpec((D, D), lambda i: (0, 0)),
      ],
      out_specs=pl.BlockSpec((BLK, D), lambda i: (i, 0)),
      out_shape=jax.ShapeDtypeStruct((NPAD, D), jnp.float32),
  )(parts, dis, b, w)


def _tc_last_body(p_ref, dis_ref, b_ref, o_ref):
  o_ref[...] = dis_ref[:, 0:1] * (p_ref[0] + p_ref[1]) + b_ref[...]


def _tc_last(parts, dis, b):
  return pl.pallas_call(
      _tc_last_body,
      grid=(_GRID,),
      in_specs=[
          pl.BlockSpec((NC, BLK, D), lambda i: (0, i, 0)),
          pl.BlockSpec((BLK, WDEG), lambda i: (i, 0)),
          pl.BlockSpec((1, D), lambda i: (0, 0)),
      ],
      out_specs=pl.BlockSpec((BLK, D), lambda i: (i, 0)),
      out_shape=jax.ShapeDtypeStruct((NPAD, D), jnp.float32),
  )(parts, dis, b)


def kernel(x, adj_t, W1, b1, W2, b2, W3, b3):
  src = jnp.concatenate(
      [adj_t[0], jnp.full((EPAD - N_EDGES,), N_NODES, jnp.int32)])
  dst = jnp.concatenate(
      [adj_t[1], jnp.full((EPAD - N_EDGES,), N_NODES, jnp.int32)])
  x_p = jnp.pad(x, ((0, NPAD - N_NODES), (0, 0)))
  ones_deg = jnp.ones((CH, WDEG), jnp.float32)
  zero_deg = jnp.zeros((RZ, WDEG), jnp.float32)
  zero_rows = jnp.zeros((RZ, D), jnp.float32)
  b1r, b2r, b3r = (b.reshape(1, D) for b in (b1, b2, b3))

  _sc_degree, _sc_prop = _sc_kernels()
  deg_parts = _sc_degree(dst, ones_deg, zero_deg)
  h, dis = _tc_first(x_p, W1, deg_parts)
  p = _sc_prop(h, src, dst, zero_rows)
  h = _tc_mid(p, dis, b1r, W2)
  p = _sc_prop(h, src, dst, zero_rows)
  h = _tc_mid(p, dis, b2r, W3)
  p = _sc_prop(h, src, dst, zero_rows)
  out = _tc_last(p, dis, b3r)
  return out[:N_NODES]


# speedup vs baseline: 5.8257x; 1.0650x over previous
import functools

import jax
import jax.numpy as jnp
from jax import lax
from jax.experimental import pallas as pl
from jax.experimental.pallas import tpu as pltpu
from jax.experimental.pallas import tpu_sc as plsc

N_NODES = 10000
N_EDGES = 320000
D = 128

NC = 2
NS = 16
NW = NC * NS

NPAD = 10240
EPAD = 327680
EPT = EPAD // NW
CH = 64
NCHUNK = EPT // CH
NB = 4
RPT = NPAD // NS
RZ = 128
WDEG = 128


def _zero_acc(zero_hbm, acc, sid):
  for r in range(RPT // RZ):
    pltpu.sync_copy(zero_hbm, acc.at[pl.ds(sid * RPT + r * RZ, RZ)])


def _read_acc(acc, out_hbm, cid, sid):
  for r in range(RPT // RZ):
    rows = pl.ds(sid * RPT + r * RZ, RZ)
    pltpu.sync_copy(acc.at[rows], out_hbm.at[cid, rows])


def _sc_degree_body(idx_hbm, ones_hbm, zero_hbm, out_hbm, acc, didx_v, ones_v):
  cid = lax.axis_index("c")
  sid = lax.axis_index("s")
  wid = cid * NS + sid
  _zero_acc(zero_hbm, acc, sid)
  pltpu.sync_copy(ones_hbm, ones_v)
  pltpu.sync_copy(idx_hbm.at[pl.ds(wid * 2 * NCHUNK, 2 * NCHUNK)], didx_v)
  plsc.subcore_barrier()

  @pl.loop(0, NCHUNK)
  def _(j):
    pltpu.sync_copy(ones_v, acc.at[didx_v.at[2 * j + 1]], add=True)

  plsc.subcore_barrier()
  _read_acc(acc, out_hbm, cid, sid)


def _sc_prop_body(h_hbm, idx_hbm, zero_hbm, out_hbm, acc, idx_v, rows_v,
                  *sems):
  gsems, isems = sems[:NB], sems[NB:]
  cid = lax.axis_index("c")
  sid = lax.axis_index("s")
  wid = cid * NS + sid
  _zero_acc(zero_hbm, acc, sid)
  plsc.subcore_barrier()

  def idx_start(j, b):
    pltpu.async_copy(idx_hbm.at[pl.ds(wid * 2 * NCHUNK + 2 * j, 2)],
                     idx_v.at[pl.ds(2 * b, 2)], isems[b])

  def idx_wait(j, b):
    pltpu.make_async_copy(idx_hbm.at[pl.ds(0, 2)],
                          idx_v.at[pl.ds(2 * b, 2)], isems[b]).wait()

  def gather_start(j, b):
    pltpu.async_copy(h_hbm.at[idx_v.at[2 * b]], rows_v.at[b], gsems[b])

  def gather_wait(j, b):
    pltpu.make_async_copy(h_hbm.at[idx_v.at[2 * b]], rows_v.at[b],
                          gsems[b]).wait()

  def scatter(j, b):
    gather_wait(j, b)
    pltpu.sync_copy(rows_v.at[b], acc.at[idx_v.at[2 * b + 1]], add=True)

  for b in range(NB):
    idx_start(b, b)
  for j in range(2):
    idx_wait(j, j)
    gather_start(j, j)

  steady = (NCHUNK - NB) // NB

  @pl.loop(0, steady)
  def _(jo):
    for b in range(NB):
      j = jo * NB + b
      b2 = (b + 2) % NB
      idx_wait(j + 2, b2)
      gather_start(j + 2, b2)
      scatter(j, b)
      idx_start(j + NB, b)

  for j in range(steady * NB, NCHUNK):
    b = j % NB
    if j + 2 < NCHUNK:
      b2 = (j + 2) % NB
      idx_wait(j + 2, b2)
      gather_start(j + 2, b2)
    scatter(j, b)

  plsc.subcore_barrier()
  _read_acc(acc, out_hbm, cid, sid)


@functools.cache
def _sc_kernels():
  mesh = plsc.VectorSubcoreMesh(
      core_axis_name="c", subcore_axis_name="s",
      num_cores=NC, num_subcores=NS)
  sc_degree = pl.kernel(
      _sc_degree_body,
      out_type=jax.ShapeDtypeStruct((NC, NPAD, WDEG), jnp.float32),
      mesh=mesh,
      scratch_types=[
          pltpu.VMEM_SHARED((NPAD, WDEG), jnp.float32),
          pltpu.VMEM((2 * NCHUNK, CH), jnp.int32),
          pltpu.VMEM((CH, WDEG), jnp.float32),
      ])
  sc_prop = pl.kernel(
      _sc_prop_body,
      out_type=jax.ShapeDtypeStruct((NC, NPAD, D), jnp.float32),
      mesh=mesh,
      scratch_types=[
          pltpu.VMEM_SHARED((NPAD, D), jnp.float32),
          pltpu.VMEM((2 * NB, CH), jnp.int32),
          pltpu.VMEM((NB, CH, D), jnp.float32),
      ] + [pltpu.SemaphoreType.DMA] * (2 * NB))
  return sc_degree, sc_prop


BLK = 1024
_GRID = NPAD // BLK


def _dis_of(deg_blk):
  return jnp.where(deg_blk > 0.0,
                   lax.rsqrt(jnp.maximum(deg_blk, 1e-12)), 0.0)


def _tc_first_body(x_ref, w_ref, degp_ref, h_ref, dis_ref):
  deg = degp_ref[0] + degp_ref[1]
  dis = _dis_of(deg)
  h = jnp.dot(x_ref[...], w_ref[...], preferred_element_type=jnp.float32)
  h_ref[...] = dis[:, 0:1] * h
  dis_ref[...] = dis


def _tc_first(x, w, deg_parts):
  return pl.pallas_call(
      _tc_first_body,
      grid=(_GRID,),
      in_specs=[
          pl.BlockSpec((BLK, D), lambda i: (i, 0)),
          pl.BlockSpec((D, D), lambda i: (0, 0)),
          pl.BlockSpec((NC, BLK, WDEG), lambda i: (0, i, 0)),
      ],
      out_specs=[
          pl.BlockSpec((BLK, D), lambda i: (i, 0)),
          pl.BlockSpec((BLK, WDEG), lambda i: (i, 0)),
      ],
      out_shape=[
          jax.ShapeDtypeStruct((NPAD, D), jnp.float32),
          jax.ShapeDtypeStruct((NPAD, WDEG), jnp.float32),
      ],
  )(x, w, deg_parts)


def _tc_mid_body(p_ref, dis_ref, b_ref, w_ref, h_ref):
  d = dis_ref[:, 0:1]
  y = jnp.maximum(d * (p_ref[0] + p_ref[1]) + b_ref[...], 0.0)
  h_ref[...] = d * jnp.dot(y, w_ref[...], preferred_element_type=jnp.float32)


def _tc_mid(parts, dis, b, w):
  return pl.pallas_call(
      _tc_mid_body,
      grid=(_GRID,),
      in_specs=[
          pl.BlockSpec((NC, BLK, D), lambda i: (0, i, 0)),
          pl.BlockSpec((BLK, WDEG), lambda i: (i, 0)),
          pl.BlockSpec((1, D), lambda i: (0, 0)),
          pl.BlockSpec((D, D), lambda i: (0, 0)),
      ],
      out_specs=pl.BlockSpec((BLK, D), lambda i: (i, 0)),
      out_shape=jax.ShapeDtypeStruct((NPAD, D), jnp.float32),
  )(parts, dis, b, w)


def _tc_last_body(p_ref, dis_ref, b_ref, o_ref):
  o_ref[...] = dis_ref[:, 0:1] * (p_ref[0] + p_ref[1]) + b_ref[...]


def _tc_last(parts, dis, b):
  return pl.pallas_call(
      _tc_last_body,
      grid=(_GRID,),
      in_specs=[
          pl.BlockSpec((NC, BLK, D), lambda i: (0, i, 0)),
          pl.BlockSpec((BLK, WDEG), lambda i: (i, 0)),
          pl.BlockSpec((1, D), lambda i: (0, 0)),
      ],
      out_specs=pl.BlockSpec((BLK, D), lambda i: (i, 0)),
      out_shape=jax.ShapeDtypeStruct((NPAD, D), jnp.float32),
  )(parts, dis, b)


def kernel(x, adj_t, W1, b1, W2, b2, W3, b3):
  src = jnp.concatenate(
      [adj_t[0], jnp.full((EPAD - N_EDGES,), N_NODES, jnp.int32)])
  dst = jnp.concatenate(
      [adj_t[1], jnp.full((EPAD - N_EDGES,), N_NODES, jnp.int32)])
  idx4 = jnp.stack(
      [src.reshape(NW, NCHUNK, CH), dst.reshape(NW, NCHUNK, CH)],
      axis=2).reshape(NW * 2 * NCHUNK, CH)
  x_p = jnp.pad(x, ((0, NPAD - N_NODES), (0, 0)))
  ones_deg = jnp.ones((CH, WDEG), jnp.float32)
  zero_rows = jnp.zeros((RZ, D), jnp.float32)
  b1r, b2r, b3r = (b.reshape(1, D) for b in (b1, b2, b3))

  _sc_degree, _sc_prop = _sc_kernels()
  deg_parts = _sc_degree(idx4, ones_deg, zero_rows)
  h, dis = _tc_first(x_p, W1, deg_parts)
  p = _sc_prop(h, idx4, zero_rows)
  h = _tc_mid(p, dis, b1r, W2)
  p = _sc_prop(h, idx4, zero_rows)
  h = _tc_mid(p, dis, b2r, W3)
  p = _sc_prop(h, idx4, zero_rows)
  out = _tc_last(p, dis, b3r)
  return out[:N_NODES]


# speedup vs baseline: 18.9235x; 3.2483x over previous
import functools

import jax
import jax.numpy as jnp
from jax import lax
from jax.experimental import pallas as pl
from jax.experimental.pallas import tpu as pltpu
from jax.experimental.pallas import tpu_sc as plsc

N_NODES = 10000
N_EDGES = 320000
D = 128

NC = 2
NS = 16
NW = NC * NS

NPAD = 10240
EPAD = 327680
EPT = EPAD // NW
CH = 64
NCHUNK = EPT // CH
NB = 4
RPT = NPAD // NS
RZ = 128
WDEG = 128


def _zero_acc(zero_hbm, acc, sid):
  for r in range(RPT // RZ):
    pltpu.sync_copy(zero_hbm, acc.at[pl.ds(sid * RPT + r * RZ, RZ)])


def _read_acc(acc, out_hbm, cid, sid):
  for r in range(RPT // RZ):
    rows = pl.ds(sid * RPT + r * RZ, RZ)
    pltpu.sync_copy(acc.at[rows], out_hbm.at[cid, rows])


def _sc_degree_body(idx_hbm, ones_hbm, zero_hbm, out_hbm, acc, didx_v, ones_v):
  cid = lax.axis_index("c")
  sid = lax.axis_index("s")
  wid = cid * NS + sid
  _zero_acc(zero_hbm, acc, sid)
  pltpu.sync_copy(ones_hbm, ones_v)
  pltpu.sync_copy(idx_hbm.at[pl.ds(wid * 2 * NCHUNK, 2 * NCHUNK)], didx_v)
  plsc.subcore_barrier()

  @pl.loop(0, NCHUNK)
  def _(j):
    pltpu.sync_copy(ones_v, acc.at[didx_v.at[2 * j + 1]], add=True)

  plsc.subcore_barrier()
  _read_acc(acc, out_hbm, cid, sid)


def _sc_prop_body(h_hbm, idx_hbm, zero_hbm, out_hbm, acc, idx_v, rows_v,
                  *sems):
  gsems, isems = sems[:NB], sems[NB:]
  cid = lax.axis_index("c")
  sid = lax.axis_index("s")
  wid = cid * NS + sid
  _zero_acc(zero_hbm, acc, sid)
  plsc.subcore_barrier()

  def idx_start(j, b):
    pltpu.async_copy(idx_hbm.at[pl.ds(wid * 2 * NCHUNK + 2 * j, 2)],
                     idx_v.at[pl.ds(2 * b, 2)], isems[b])

  def idx_wait(j, b):
    pltpu.make_async_copy(idx_hbm.at[pl.ds(0, 2)],
                          idx_v.at[pl.ds(2 * b, 2)], isems[b]).wait()

  def gather_start(j, b):
    pltpu.async_copy(h_hbm.at[idx_v.at[2 * b]], rows_v.at[b], gsems[b])

  def gather_wait(j, b):
    pltpu.make_async_copy(h_hbm.at[idx_v.at[2 * b]], rows_v.at[b],
                          gsems[b]).wait()

  def scatter(j, b):
    gather_wait(j, b)
    pltpu.sync_copy(rows_v.at[b], acc.at[idx_v.at[2 * b + 1]], add=True)

  for b in range(NB):
    idx_start(b, b)
  for j in range(2):
    idx_wait(j, j)
    gather_start(j, j)

  steady = (NCHUNK - NB) // NB

  @pl.loop(0, steady)
  def _(jo):
    for b in range(NB):
      j = jo * NB + b
      b2 = (b + 2) % NB
      idx_wait(j + 2, b2)
      gather_start(j + 2, b2)
      scatter(j, b)
      idx_start(j + NB, b)

  for j in range(steady * NB, NCHUNK):
    b = j % NB
    if j + 2 < NCHUNK:
      b2 = (j + 2) % NB
      idx_wait(j + 2, b2)
      gather_start(j + 2, b2)
    scatter(j, b)

  plsc.subcore_barrier()
  _read_acc(acc, out_hbm, cid, sid)


@functools.cache
def _sc_kernels():
  mesh = plsc.VectorSubcoreMesh(
      core_axis_name="c", subcore_axis_name="s",
      num_cores=NC, num_subcores=NS)
  sc_degree = pl.kernel(
      _sc_degree_body,
      out_type=jax.ShapeDtypeStruct((NC, NPAD, WDEG), jnp.float32),
      mesh=mesh,
      scratch_types=[
          pltpu.VMEM_SHARED((NPAD, WDEG), jnp.float32),
          pltpu.VMEM((2 * NCHUNK, CH), jnp.int32),
          pltpu.VMEM((CH, WDEG), jnp.float32),
      ])
  sc_prop = pl.kernel(
      _sc_prop_body,
      out_type=jax.ShapeDtypeStruct((NC, NPAD, D), jnp.float32),
      mesh=mesh,
      scratch_types=[
          pltpu.VMEM_SHARED((NPAD, D), jnp.float32),
          pltpu.VMEM((2 * NB, CH), jnp.int32),
          pltpu.VMEM((NB, CH, D), jnp.float32),
      ] + [pltpu.SemaphoreType.DMA] * (2 * NB))
  return sc_degree, sc_prop


BLK = 1024
_GRID = NPAD // BLK


def _dis_of(deg_blk):
  return jnp.where(deg_blk > 0.0,
                   lax.rsqrt(jnp.maximum(deg_blk, 1e-12)), 0.0)


def _tc_first_body(x_ref, w_ref, degp_ref, h_ref, dis_ref):
  deg = degp_ref[0] + degp_ref[1]
  dis = _dis_of(deg)
  h = jnp.dot(x_ref[...], w_ref[...], preferred_element_type=jnp.float32)
  h_ref[...] = dis[:, 0:1] * h
  dis_ref[...] = dis


def _tc_first(x, w, deg_parts):
  return pl.pallas_call(
      _tc_first_body,
      grid=(_GRID,),
      in_specs=[
          pl.BlockSpec((BLK, D), lambda i: (i, 0)),
          pl.BlockSpec((D, D), lambda i: (0, 0)),
          pl.BlockSpec((NC, BLK, WDEG), lambda i: (0, i, 0)),
      ],
      out_specs=[
          pl.BlockSpec((BLK, D), lambda i: (i, 0)),
          pl.BlockSpec((BLK, WDEG), lambda i: (i, 0)),
      ],
      out_shape=[
          jax.ShapeDtypeStruct((NPAD, D), jnp.float32),
          jax.ShapeDtypeStruct((NPAD, WDEG), jnp.float32),
      ],
  )(x, w, deg_parts)


def _tc_mid_body(p_ref, dis_ref, b_ref, w_ref, h_ref):
  d = dis_ref[:, 0:1]
  y = jnp.maximum(d * (p_ref[0] + p_ref[1]) + b_ref[...], 0.0)
  h_ref[...] = d * jnp.dot(y, w_ref[...], preferred_element_type=jnp.float32)


def _tc_mid(parts, dis, b, w):
  return pl.pallas_call(
      _tc_mid_body,
      grid=(_GRID,),
      in_specs=[
          pl.BlockSpec((NC, BLK, D), lambda i: (0, i, 0)),
          pl.BlockSpec((BLK, WDEG), lambda i: (i, 0)),
          pl.BlockSpec((1, D), lambda i: (0, 0)),
          pl.BlockSpec((D, D), lambda i: (0, 0)),
      ],
      out_specs=pl.BlockSpec((BLK, D), lambda i: (i, 0)),
      out_shape=jax.ShapeDtypeStruct((NPAD, D), jnp.float32),
  )(parts, dis, b, w)


def _tc_last_body(p_ref, dis_ref, b_ref, o_ref):
  o_ref[...] = dis_ref[:, 0:1] * (p_ref[0] + p_ref[1]) + b_ref[...]


def _tc_last(parts, dis, b):
  return pl.pallas_call(
      _tc_last_body,
      grid=(_GRID,),
      in_specs=[
          pl.BlockSpec((NC, BLK, D), lambda i: (0, i, 0)),
          pl.BlockSpec((BLK, WDEG), lambda i: (i, 0)),
          pl.BlockSpec((1, D), lambda i: (0, 0)),
      ],
      out_specs=pl.BlockSpec((BLK, D), lambda i: (i, 0)),
      out_shape=jax.ShapeDtypeStruct((NPAD, D), jnp.float32),
  )(parts, dis, b)


def kernel(x, adj_t, W1, b1, W2, b2, W3, b3):
  pad_idx = N_NODES + jnp.arange(EPAD - N_EDGES, dtype=jnp.int32) % (
      NPAD - N_NODES)
  src = jnp.concatenate([adj_t[0], pad_idx])
  dst = jnp.concatenate([adj_t[1], pad_idx])
  idx4 = jnp.stack(
      [src.reshape(NW, NCHUNK, CH), dst.reshape(NW, NCHUNK, CH)],
      axis=2).reshape(NW * 2 * NCHUNK, CH)
  x_p = jnp.pad(x, ((0, NPAD - N_NODES), (0, 0)))
  ones_deg = jnp.ones((CH, WDEG), jnp.float32)
  zero_rows = jnp.zeros((RZ, D), jnp.float32)
  b1r, b2r, b3r = (b.reshape(1, D) for b in (b1, b2, b3))

  _sc_degree, _sc_prop = _sc_kernels()
  deg_parts = _sc_degree(idx4, ones_deg, zero_rows)
  h, dis = _tc_first(x_p, W1, deg_parts)
  p = _sc_prop(h, idx4, zero_rows)
  h = _tc_mid(p, dis, b1r, W2)
  p = _sc_prop(h, idx4, zero_rows)
  h = _tc_mid(p, dis, b2r, W3)
  p = _sc_prop(h, idx4, zero_rows)
  out = _tc_last(p, dis, b3r)
  return out[:N_NODES]


# speedup vs baseline: 21.8577x; 1.1551x over previous
import functools

import jax
import jax.numpy as jnp
from jax import lax
from jax.experimental import pallas as pl
from jax.experimental.pallas import tpu as pltpu
from jax.experimental.pallas import tpu_sc as plsc

N_NODES = 10000
N_EDGES = 320000
D = 128

NC = 2
NS = 16
NW = NC * NS

NPAD = 10240
EPAD = 327680
EPT = EPAD // NW
CH = 64
NCHUNK = EPT // CH
NB = 4
RPT = NPAD // NS
RZ = 128
WDEG = 8


def _zero_acc(zero_hbm, acc, sid):
  for r in range(RPT // RZ):
    pltpu.sync_copy(zero_hbm, acc.at[pl.ds(sid * RPT + r * RZ, RZ)])


def _read_acc(acc, out_hbm, cid, sid):
  for r in range(RPT // RZ):
    rows = pl.ds(sid * RPT + r * RZ, RZ)
    pltpu.sync_copy(acc.at[rows], out_hbm.at[cid, rows])


def _sc_degree_body(dst_hbm, out_hbm, stage, hist_v, didx_v, red_v, bcast_v):
  cid = lax.axis_index("c")
  sid = lax.axis_index("s")
  wid = cid * NS + sid

  @pl.loop(0, NPAD // 16)
  def _(k):
    hist_v[pl.ds(k * 16, 16)] = jnp.zeros((16,), jnp.float32)

  pltpu.sync_copy(dst_hbm.at[pl.ds(wid * EPT, EPT)], didx_v)
  ones16 = jnp.ones((16,), jnp.float32)

  @pl.loop(0, EPT // 16)
  def _(j):
    plsc.addupdate_scatter(hist_v, [didx_v[pl.ds(j * 16, 16)]], ones16)

  pltpu.sync_copy(hist_v, stage.at[sid])
  plsc.subcore_barrier()
  for t in range(NS):
    pltpu.sync_copy(stage.at[t, pl.ds(sid * RPT, RPT)], red_v.at[t])

  @pl.loop(0, RPT // 16)
  def _(k):
    s = red_v[0, pl.ds(k * 16, 16)]
    for t in range(1, NS):
      s = s + red_v[t, pl.ds(k * 16, 16)]
    hist_v[pl.ds(k * 16, 16)] = s

  lane = lax.iota(jnp.int32, 16) // WDEG

  @pl.loop(0, RPT * WDEG // 16)
  def _(m):
    vals = plsc.load_gather(hist_v, [m * (16 // WDEG) + lane])
    bcast_v[pl.ds(m * 16, 16)] = vals

  pltpu.sync_copy(bcast_v,
                  out_hbm.at[cid, pl.ds(sid * RPT * WDEG, RPT * WDEG)])


def _sc_prop_body(h_hbm, idx_hbm, zero_hbm, out_hbm, acc, idx_v, rows_v,
                  *sems):
  gsems, isems = sems[:NB], sems[NB:]
  cid = lax.axis_index("c")
  sid = lax.axis_index("s")
  wid = cid * NS + sid
  _zero_acc(zero_hbm, acc, sid)
  plsc.subcore_barrier()

  def idx_start(j, b):
    pltpu.async_copy(idx_hbm.at[pl.ds(wid * 2 * NCHUNK + 2 * j, 2)],
                     idx_v.at[pl.ds(2 * b, 2)], isems[b])

  def idx_wait(j, b):
    pltpu.make_async_copy(idx_hbm.at[pl.ds(0, 2)],
                          idx_v.at[pl.ds(2 * b, 2)], isems[b]).wait()

  def gather_start(j, b):
    pltpu.async_copy(h_hbm.at[idx_v.at[2 * b]], rows_v.at[b], gsems[b])

  def gather_wait(j, b):
    pltpu.make_async_copy(h_hbm.at[idx_v.at[2 * b]], rows_v.at[b],
                          gsems[b]).wait()

  def scatter(j, b):
    gather_wait(j, b)
    pltpu.sync_copy(rows_v.at[b], acc.at[idx_v.at[2 * b + 1]], add=True)

  for b in range(NB):
    idx_start(b, b)
  for j in range(2):
    idx_wait(j, j)
    gather_start(j, j)

  steady = (NCHUNK - NB) // NB

  @pl.loop(0, steady)
  def _(jo):
    for b in range(NB):
      j = jo * NB + b
      b2 = (b + 2) % NB
      idx_wait(j + 2, b2)
      gather_start(j + 2, b2)
      scatter(j, b)
      idx_start(j + NB, b)

  for j in range(steady * NB, NCHUNK):
    b = j % NB
    if j + 2 < NCHUNK:
      b2 = (j + 2) % NB
      idx_wait(j + 2, b2)
      gather_start(j + 2, b2)
    scatter(j, b)

  plsc.subcore_barrier()
  _read_acc(acc, out_hbm, cid, sid)


@functools.cache
def _sc_kernels():
  mesh = plsc.VectorSubcoreMesh(
      core_axis_name="c", subcore_axis_name="s",
      num_cores=NC, num_subcores=NS)
  sc_degree = pl.kernel(
      _sc_degree_body,
      out_type=jax.ShapeDtypeStruct((NC, NPAD * WDEG), jnp.float32),
      mesh=mesh,
      compiler_params=pltpu.CompilerParams(needs_layout_passes=False),
      scratch_types=[
          pltpu.VMEM_SHARED((NS, NPAD), jnp.float32),
          pltpu.VMEM((NPAD,), jnp.float32),
          pltpu.VMEM((EPT,), jnp.int32),
          pltpu.VMEM((NS, RPT), jnp.float32),
          pltpu.VMEM((RPT * WDEG,), jnp.float32),
      ])
  sc_prop = pl.kernel(
      _sc_prop_body,
      out_type=jax.ShapeDtypeStruct((NC, NPAD, D), jnp.float32),
      mesh=mesh,
      scratch_types=[
          pltpu.VMEM_SHARED((NPAD, D), jnp.float32),
          pltpu.VMEM((2 * NB, CH), jnp.int32),
          pltpu.VMEM((NB, CH, D), jnp.float32),
      ] + [pltpu.SemaphoreType.DMA] * (2 * NB))
  return sc_degree, sc_prop


BLK = 1024
_GRID = NPAD // BLK


def _dis_of(deg_blk):
  return jnp.where(deg_blk > 0.0,
                   lax.rsqrt(jnp.maximum(deg_blk, 1e-12)), 0.0)


def _tc_first_body(x_ref, w_ref, degp_ref, h_ref, dis_ref):
  deg = degp_ref[0] + degp_ref[1]
  dis = _dis_of(deg)
  h = jnp.dot(x_ref[...], w_ref[...], preferred_element_type=jnp.float32)
  h_ref[...] = dis[:, 0:1] * h
  dis_ref[...] = dis


def _tc_first(x, w, deg_parts):
  return pl.pallas_call(
      _tc_first_body,
      grid=(_GRID,),
      in_specs=[
          pl.BlockSpec((BLK, D), lambda i: (i, 0)),
          pl.BlockSpec((D, D), lambda i: (0, 0)),
          pl.BlockSpec((NC, BLK, WDEG), lambda i: (0, i, 0)),
      ],
      out_specs=[
          pl.BlockSpec((BLK, D), lambda i: (i, 0)),
          pl.BlockSpec((BLK, WDEG), lambda i: (i, 0)),
      ],
      out_shape=[
          jax.ShapeDtypeStruct((NPAD, D), jnp.float32),
          jax.ShapeDtypeStruct((NPAD, WDEG), jnp.float32),
      ],
  )(x, w, deg_parts)


def _tc_mid_body(p_ref, dis_ref, b_ref, w_ref, h_ref):
  d = dis_ref[:, 0:1]
  y = jnp.maximum(d * (p_ref[0] + p_ref[1]) + b_ref[...], 0.0)
  h_ref[...] = d * jnp.dot(y, w_ref[...], preferred_element_type=jnp.float32)


def _tc_mid(parts, dis, b, w):
  return pl.pallas_call(
      _tc_mid_body,
      grid=(_GRID,),
      in_specs=[
          pl.BlockSpec((NC, BLK, D), lambda i: (0, i, 0)),
          pl.BlockSpec((BLK, WDEG), lambda i: (i, 0)),
          pl.BlockSpec((1, D), lambda i: (0, 0)),
          pl.BlockSpec((D, D), lambda i: (0, 0)),
      ],
      out_specs=pl.BlockSpec((BLK, D), lambda i: (i, 0)),
      out_shape=jax.ShapeDtypeStruct((NPAD, D), jnp.float32),
  )(parts, dis, b, w)


def _tc_last_body(p_ref, dis_ref, b_ref, o_ref):
  o_ref[...] = dis_ref[:, 0:1] * (p_ref[0] + p_ref[1]) + b_ref[...]


def _tc_last(parts, dis, b):
  return pl.pallas_call(
      _tc_last_body,
      grid=(_GRID,),
      in_specs=[
          pl.BlockSpec((NC, BLK, D), lambda i: (0, i, 0)),
          pl.BlockSpec((BLK, WDEG), lambda i: (i, 0)),
          pl.BlockSpec((1, D), lambda i: (0, 0)),
      ],
      out_specs=pl.BlockSpec((BLK, D), lambda i: (i, 0)),
      out_shape=jax.ShapeDtypeStruct((NPAD, D), jnp.float32),
  )(parts, dis, b)


def kernel(x, adj_t, W1, b1, W2, b2, W3, b3):
  pad_idx = N_NODES + jnp.arange(EPAD - N_EDGES, dtype=jnp.int32) % (
      NPAD - N_NODES)
  src = jnp.concatenate([adj_t[0], pad_idx])
  dst = jnp.concatenate([adj_t[1], pad_idx])
  idx4 = jnp.stack(
      [src.reshape(NW, NCHUNK, CH), dst.reshape(NW, NCHUNK, CH)],
      axis=2).reshape(NW * 2 * NCHUNK, CH)
  x_p = jnp.pad(x, ((0, NPAD - N_NODES), (0, 0)))
  zero_rows = jnp.zeros((RZ, D), jnp.float32)
  b1r, b2r, b3r = (b.reshape(1, D) for b in (b1, b2, b3))

  _sc_degree, _sc_prop = _sc_kernels()
  deg_parts = _sc_degree(dst).reshape(NC, NPAD, WDEG)
  h, dis = _tc_first(x_p, W1, deg_parts)
  p = _sc_prop(h, idx4, zero_rows)
  h = _tc_mid(p, dis, b1r, W2)
  p = _sc_prop(h, idx4, zero_rows)
  h = _tc_mid(p, dis, b2r, W3)
  p = _sc_prop(h, idx4, zero_rows)
  out = _tc_last(p, dis, b3r)
  return out[:N_NODES]
